# baseline (speedup 1.0000x reference)
import functools

import jax
import jax.numpy as jnp
from jax import lax
from jax.experimental import pallas as pl
from jax.experimental.pallas import tpu as pltpu
from jax.experimental.pallas import tpu_sc as plsc

RADIUS = 0.2
K = 32
EPS = 1e-5

NC = 2
NS = 16
L = 16
NW = NC * NS


def _mm_stats_body(x_ref, w_ref, y_ref, st_ref):
    x = x_ref[...]
    y = jnp.dot(x, w_ref[...], preferred_element_type=jnp.float32)
    y_ref[...] = y
    st = jnp.concatenate(
        [jnp.sum(y, axis=0, keepdims=True),
         jnp.sum(y * y, axis=0, keepdims=True)], axis=0)

    @pl.when(pl.program_id(0) == 0)
    def _():
        st_ref[...] = st

    @pl.when(pl.program_id(0) != 0)
    def _():
        st_ref[...] = st_ref[...] + st


def _mm_stats(flat_feats, w_fa):
    bn_tot, ci = flat_feats.shape
    cw = w_fa.shape[1]
    blk = 2048
    grid = bn_tot // blk
    return pl.pallas_call(
        _mm_stats_body,
        grid=(grid,),
        in_specs=[
            pl.BlockSpec((blk, ci), lambda i: (i, 0)),
            pl.BlockSpec((ci, cw), lambda i: (0, 0)),
        ],
        out_specs=[
            pl.BlockSpec((blk, cw), lambda i: (i, 0)),
            pl.BlockSpec((2, cw), lambda i: (0, 0)),
        ],
        out_shape=[
            jax.ShapeDtypeStruct((bn_tot, cw), jnp.float32),
            jax.ShapeDtypeStruct((2, cw), jnp.float32),
        ],
    )(flat_feats, w_fa)


def _ball_query_sc(xs, ys, zs, cxs, cys, czs, B, N, M):
    BM = B * M
    CPT = BM // NW
    TPB = NW // B
    MPT = M // TPB
    nchunk = N // L
    r2 = jnp.float32(RADIUS * RADIUS)

    mesh = plsc.VectorSubcoreMesh(
        core_axis_name="c", subcore_axis_name="s", num_cores=NC,
        num_subcores=NS)

    @functools.partial(
        pl.kernel,
        mesh=mesh,
        compiler_params=pltpu.CompilerParams(needs_layout_passes=False),
        out_type=[
            jax.ShapeDtypeStruct((BM * K,), jnp.int32),
            jax.ShapeDtypeStruct((BM,), jnp.int32),
            jax.ShapeDtypeStruct((BM * 3 * K,), jnp.float32),
            jax.ShapeDtypeStruct((NW * 16 * L,), jnp.float32),
        ],
        scratch_types=[
            pltpu.VMEM((N,), jnp.float32),
            pltpu.VMEM((N,), jnp.float32),
            pltpu.VMEM((N,), jnp.float32),
            pltpu.VMEM((MPT + L,), jnp.float32),
            pltpu.VMEM((MPT + L,), jnp.float32),
            pltpu.VMEM((MPT + L,), jnp.float32),
            pltpu.VMEM((K + L,), jnp.int32),
            pltpu.VMEM((CPT * K,), jnp.int32),
            pltpu.VMEM((CPT,), jnp.int32),
            pltpu.VMEM((CPT * 3 * K,), jnp.float32),
            pltpu.VMEM((16 * L,), jnp.float32),
            pltpu.VMEM((N,), jnp.int32),
            pltpu.VMEM((N,), jnp.int32),
        ],
    )
    def kb(x_hbm, y_hbm, z_hbm, cx_hbm, cy_hbm, cz_hbm,
           idx_out, cnt_out, off_out, part_out,
           x_v, y_v, z_v, cx_v, cy_v, cz_v, ib_v, idx_t, cnt_t, off_t, acc_v,
           cand_v, cnts_v):
        wid = lax.axis_index("s") * NC + lax.axis_index("c")
        b = wid // TPB
        mb = b * M + (wid % TPB) * MPT
        pltpu.sync_copy(x_hbm.at[pl.ds(b * N, N)], x_v)
        pltpu.sync_copy(y_hbm.at[pl.ds(b * N, N)], y_v)
        pltpu.sync_copy(z_hbm.at[pl.ds(b * N, N)], z_v)
        pltpu.sync_copy(cx_hbm.at[pl.ds(mb, MPT)], cx_v.at[pl.ds(0, MPT)])
        pltpu.sync_copy(cy_hbm.at[pl.ds(mb, MPT)], cy_v.at[pl.ds(0, MPT)])
        pltpu.sync_copy(cz_hbm.at[pl.ds(mb, MPT)], cz_v.at[pl.ds(0, MPT)])

        lanes = lax.iota(jnp.int32, L)
        for r in range(16):
            acc_v[pl.ds(r * L, L)] = jnp.zeros((L,), jnp.float32)
        row_base = jnp.int32(b * N)

        def per_centroid(i, _):
            cxi = cx_v[pl.ds(i, L)][0]
            cyi = cy_v[pl.ds(i, L)][0]
            czi = cz_v[pl.ds(i, L)][0]

            def cond(st):
                j, cv = st
                return jnp.logical_and(j < nchunk, cv[0] < K)

            def body(st):
                j, cv = st
                for u in range(4):
                    base = (j + u) * L
                    dx = x_v[pl.ds(base, L)] - cxi
                    dy = y_v[pl.ds(base, L)] - cyi
                    dz = z_v[pl.ds(base, L)] - czi
                    d2 = dx * dx + dy * dy + dz * dz
                    msk = d2 < r2
                    plsc.store_compressed(cand_v.at[pl.ds(base, L)],
                                          base + lanes, mask=msk)
                    pc = plsc.all_reduce_population_count(msk)
                    cnts_v[pl.ds(base, L)] = pc
                    cv = cv + pc
                return j + 4, cv

            jstop, cvec = lax.while_loop(
                cond, body, (jnp.int32(0), jnp.zeros((L,), jnp.int32)))
            c = jnp.minimum(cvec[0], K)

            nb = (jstop + (L - 1)) // L

            def per_block(bi, carry):
                cb = bi * L
                cidx = cb + lanes
                cnts = plsc.load_gather(cnts_v, [cidx * L])
                cnts = jnp.where(cidx < jstop, cnts, 0)
                incl = plsc.cumsum(cnts)
                excl = incl - cnts + carry
                for t in range(L):
                    pj = jnp.minimum(excl[t], K)
                    cj = cnts[t]
                    vals = cand_v[pl.ds((cb + t) * L, L)]
                    m = jnp.logical_and(lanes < cj, pj + lanes < K)
                    plsc.store_compressed(ib_v.at[pl.ds(pj, L)], vals,
                                          mask=m)
                return carry + incl[L - 1]

            lax.fori_loop(0, nb, per_block, jnp.int32(0))
            plsc.store_scatter(cnt_t, [jnp.full((L,), i, jnp.int32)],
                               jnp.full((L,), c, jnp.int32),
                               mask=lanes < 1)
            first = jnp.where(c > 0, ib_v[pl.ds(0, L)][0], 0)
            for v in range(K // L):
                ln = jnp.int32(v * L) + lanes
                vals = ib_v[pl.ds(v * L, L)]
                vals = jnp.where(ln < c, vals, first)
                ox = plsc.load_gather(x_v, [vals]) - cxi
                oy = plsc.load_gather(y_v, [vals]) - cyi
                oz = plsc.load_gather(z_v, [vals]) - czi
                ob = i * (3 * K) + v * L
                off_t[pl.ds(ob, L)] = ox
                off_t[pl.ds(ob + K, L)] = oy
                off_t[pl.ds(ob + 2 * K, L)] = oz
                acc_v[pl.ds(0 * L, L)] = acc_v[pl.ds(0 * L, L)] + ox
                acc_v[pl.ds(1 * L, L)] = acc_v[pl.ds(1 * L, L)] + oy
                acc_v[pl.ds(2 * L, L)] = acc_v[pl.ds(2 * L, L)] + oz
                acc_v[pl.ds(3 * L, L)] = acc_v[pl.ds(3 * L, L)] + ox * ox
                acc_v[pl.ds(4 * L, L)] = acc_v[pl.ds(4 * L, L)] + oy * oy
                acc_v[pl.ds(5 * L, L)] = acc_v[pl.ds(5 * L, L)] + oz * oz
                acc_v[pl.ds(6 * L, L)] = acc_v[pl.ds(6 * L, L)] + ox * oy
                acc_v[pl.ds(7 * L, L)] = acc_v[pl.ds(7 * L, L)] + ox * oz
                acc_v[pl.ds(8 * L, L)] = acc_v[pl.ds(8 * L, L)] + oy * oz
                idx_t[pl.ds(i * K + v * L, L)] = vals + row_base
            return 0

        lax.fori_loop(0, CPT, per_centroid, 0)
        pltpu.sync_copy(idx_t, idx_out.at[pl.ds(wid * (CPT * K), CPT * K)])
        pltpu.sync_copy(cnt_t, cnt_out.at[pl.ds(wid * CPT, CPT)])
        pltpu.sync_copy(off_t,
                        off_out.at[pl.ds(wid * (CPT * 3 * K), CPT * 3 * K)])
        pltpu.sync_copy(acc_v, part_out.at[pl.ds(wid * (16 * L), 16 * L)])

    return kb(xs, ys, zs, cxs, cys, czs)


def _finalize_body(n1, n2, st_ref, part_ref, gfa_ref, bfa_ref, wx_ref,
                   gx_ref, bx_ref, out_ref):
    st = st_ref[...]
    mean_fa = st[0:1, :] / n1
    var_fa = st[1:2, :] / n1 - mean_fa * mean_fa
    scale_fa = gfa_ref[...] * lax.rsqrt(var_fa + EPS)
    shift_fa = bfa_ref[...] - mean_fa * scale_fa

    p = part_ref[...]
    s = jnp.sum(jnp.sum(p, axis=0), axis=1, keepdims=True)
    w0 = wx_ref[0:1, :]
    w1 = wx_ref[1:2, :]
    w2 = wx_ref[2:3, :]
    mx = (s[0, 0] * w0 + s[1, 0] * w1 + s[2, 0] * w2) / n2
    exx = (s[3, 0] * w0 * w0 + s[4, 0] * w1 * w1 + s[5, 0] * w2 * w2
           + 2.0 * (s[6, 0] * w0 * w1 + s[7, 0] * w0 * w2
                    + s[8, 0] * w1 * w2)) / n2
    var_x = exx - mx * mx
    scale_x = gx_ref[...] * lax.rsqrt(var_x + EPS)
    shift_x = bx_ref[...] - mx * scale_x

    pad = jnp.zeros_like(w0)
    xrows = jnp.concatenate(
        [jnp.concatenate([r * scale_x, pad], axis=1)
         for r in (w0, w1, w2)] + [jnp.concatenate([shift_x, pad], axis=1)],
        axis=0)
    out_ref[...] = jnp.concatenate(
        [scale_fa, shift_fa, xrows,
         jnp.zeros((2, scale_fa.shape[1]), jnp.float32)], axis=0)


def _finalize(st, part, g_fa, b_fa, w_xyz, g_xyz, b_xyz, n1, n2):
    cw = st.shape[1]
    c0 = w_xyz.shape[1]
    return pl.pallas_call(
        functools.partial(_finalize_body, float(n1), float(n2)),
        out_shape=jax.ShapeDtypeStruct((8, cw), jnp.float32),
    )(st, part, g_fa.reshape(1, cw), b_fa.reshape(1, cw), w_xyz,
      g_xyz.reshape(1, c0), b_xyz.reshape(1, c0))


def _attend_sc(yfa, idx_flat, off_flat, consts, BM, C0):
    CW = yfa.shape[1]
    CPT = BM // NW
    G = 8
    NG = CPT // G
    NCV = C0 // L

    mesh = plsc.VectorSubcoreMesh(
        core_axis_name="c", subcore_axis_name="s", num_cores=NC,
        num_subcores=NS)

    @functools.partial(
        pl.kernel,
        mesh=mesh,
        compiler_params=pltpu.CompilerParams(needs_layout_passes=False),
        out_type=jax.ShapeDtypeStruct((BM * C0,), jnp.float32),
        scratch_types=[
            pltpu.VMEM((CPT * K,), jnp.int32),
            pltpu.VMEM((CPT * 3 * K,), jnp.float32),
            pltpu.VMEM((G * K, CW), jnp.float32),
            pltpu.VMEM((G * K, CW), jnp.float32),
            pltpu.VMEM((8, CW), jnp.float32),
            pltpu.VMEM((CPT * C0,), jnp.float32),
            pltpu.SemaphoreType.DMA,
            pltpu.SemaphoreType.DMA,
        ],
    )
    def kd(yfa_hbm, idx_hbm, off_hbm, c_hbm, out_hbm,
           idx_v, off_v, yb0_v, yb1_v, cc_v, out_v, sem0, sem1):
        wid = lax.axis_index("s") * NC + lax.axis_index("c")
        base = wid * CPT
        pltpu.sync_copy(idx_hbm.at[pl.ds(base * K, CPT * K)], idx_v)
        pltpu.sync_copy(off_hbm.at[pl.ds(base * 3 * K, CPT * 3 * K)], off_v)
        pltpu.sync_copy(c_hbm, cc_v)

        sc = [cc_v[0, pl.ds(cv * L, L)] for cv in range(2 * NCV)]
        sh = [cc_v[1, pl.ds(cv * L, L)] for cv in range(2 * NCV)]
        we0 = [cc_v[2, pl.ds(cv * L, L)] for cv in range(NCV)]
        we1 = [cc_v[3, pl.ds(cv * L, L)] for cv in range(NCV)]
        we2 = [cc_v[4, pl.ds(cv * L, L)] for cv in range(NCV)]
        hx = [cc_v[5, pl.ds(cv * L, L)] for cv in range(NCV)]
        GK = G * K

        def gsrc(g):
            return yfa_hbm.at[idx_v.at[pl.ds(g * GK, GK)]]

        def compute_group(g, buf):
            def per_centroid(ci, _):
                r = g * G + ci
                ob = r * (3 * K)
                dq = [off_v[pl.ds(ob + q * L, L)] for q in range(6)]
                rowb = ci * K
                nums = [jnp.zeros((L,), jnp.float32) for _ in range(NCV)]
                dens = [jnp.zeros((L,), jnp.float32) for _ in range(NCV)]
                for k in range(K):
                    q, ln = divmod(k, L)
                    dxk = dq[q][ln]
                    dyk = dq[2 + q][ln]
                    dzk = dq[4 + q][ln]
                    row = rowb + k
                    for cv in range(NCV):
                        s = pl.ds(cv * L, L)
                        sa = pl.ds(C0 + cv * L, L)
                        gxf = (dxk * we0[cv] + dyk * we1[cv]
                               + dzk * we2[cv] + hx[cv])
                        f = buf[row, s] * sc[cv] + sh[cv] + gxf
                        a = (buf[row, sa] * sc[NCV + cv]
                             + sh[NCV + cv] + gxf)
                        e = jnp.exp(f)
                        nums[cv] = nums[cv] + e * a
                        dens[cv] = dens[cv] + e
                for cv in range(NCV):
                    out_v[pl.ds(r * C0 + cv * L, L)] = nums[cv] / dens[cv]
                return 0

            lax.fori_loop(0, G, per_centroid, 0)

        pltpu.async_copy(gsrc(0), yb0_v, sem0)

        def per_pair(p, _):
            g0 = 2 * p
            pltpu.make_async_copy(gsrc(g0), yb0_v, sem0).wait()
            pltpu.async_copy(gsrc(g0 + 1), yb1_v, sem1)
            compute_group(g0, yb0_v)
            pltpu.make_async_copy(gsrc(g0 + 1), yb1_v, sem1).wait()

            @pl.when(p + 1 < NG // 2)
            def _():
                pltpu.async_copy(gsrc(g0 + 2), yb0_v, sem0)

            compute_group(g0 + 1, yb1_v)
            return 0

        lax.fori_loop(0, NG // 2, per_pair, 0)
        pltpu.sync_copy(out_v, out_hbm.at[pl.ds(base * C0, CPT * C0)])

    return kd(yfa, idx_flat, off_flat, consts)


def _mlp_body(x_ref, cnt_ref, w1_ref, g1_ref, b1_ref, w2_ref, g2_ref, b2_ref,
              out_ref):
    x = x_ref[...]
    n = x.shape[0]
    t = jnp.dot(x, w1_ref[...], preferred_element_type=jnp.float32)
    m = jnp.sum(t, axis=0, keepdims=True) / n
    v = jnp.sum(t * t, axis=0, keepdims=True) / n - m * m
    t = (t - m) * lax.rsqrt(v + EPS) * g1_ref[...] + b1_ref[...]
    t = jnp.maximum(t, 0.0)
    u = jnp.dot(t, w2_ref[...], preferred_element_type=jnp.float32)
    m2 = jnp.sum(u, axis=0, keepdims=True) / n
    v2 = jnp.sum(u * u, axis=0, keepdims=True) / n - m2 * m2
    u = (u - m2) * lax.rsqrt(v2 + EPS) * g2_ref[...] + b2_ref[...]
    u = jnp.maximum(u, 0.0)
    valid = (cnt_ref[...] > 0).astype(jnp.float32)
    out_ref[...] = u * valid


def _mlp(nf, cnt, w1, g1, b1, w2, g2, b2):
    bm, c1 = nf.shape[0], w1.shape[1]
    c2 = w2.shape[1]
    return pl.pallas_call(
        _mlp_body,
        out_shape=jax.ShapeDtypeStruct((bm, c2), jnp.float32),
    )(nf, cnt.reshape(bm, 1), w1, g1.reshape(1, c1), b1.reshape(1, c1),
      w2, g2.reshape(1, c2), b2.reshape(1, c2))


def kernel(xyz, new_xyz, feats, W_feat, g_feat, b_feat, W_attn, g_attn,
           b_attn, W_xyz, g_xyz, b_xyz, W_m1, g_m1, b_m1, W_m2, g_m2, b_m2):
    B, N, _ = xyz.shape
    M = new_xyz.shape[1]
    Ci = feats.shape[2]
    C0 = W_feat.shape[1]
    C2 = W_m2.shape[1]
    BM = B * M

    xs = xyz[:, :, 0].reshape(-1)
    ys = xyz[:, :, 1].reshape(-1)
    zs = xyz[:, :, 2].reshape(-1)
    cxs = new_xyz[:, :, 0].reshape(-1)
    cys = new_xyz[:, :, 1].reshape(-1)
    czs = new_xyz[:, :, 2].reshape(-1)

    w_fa = jnp.concatenate([W_feat, W_attn], axis=1)
    g_fa = jnp.concatenate([g_feat, g_attn], axis=0)
    b_fa = jnp.concatenate([b_feat, b_attn], axis=0)

    yfa, st = _mm_stats(feats.reshape(B * N, Ci), w_fa)
    idx, cnt, off, part = _ball_query_sc(xs, ys, zs, cxs, cys, czs, B, N, M)
    consts = _finalize(st, part.reshape(NW, 16, L), g_fa, b_fa, W_xyz,
                       g_xyz, b_xyz, B * N, BM * K)
    nf = _attend_sc(yfa, idx, off, consts, BM, C0)
    out = _mlp(nf.reshape(BM, C0), cnt, W_m1, g_m1, b_m1, W_m2, g_m2, b_m2)
    return out.reshape(B, M, C2)


# speedup vs baseline: 1.5905x; 1.5905x over previous
import functools

import jax
import jax.numpy as jnp
from jax import lax
from jax.experimental import pallas as pl
from jax.experimental.pallas import tpu as pltpu
from jax.experimental.pallas import tpu_sc as plsc

RADIUS = 0.2
K = 32
EPS = 1e-5

NC = 2
NS = 16
L = 16
NW = NC * NS


def _mm_stats_body(x_ref, w_ref, y_ref, st_ref):
    x = x_ref[...]
    y = jnp.dot(x, w_ref[...], preferred_element_type=jnp.float32)
    y_ref[...] = y
    st = jnp.concatenate(
        [jnp.sum(y, axis=0, keepdims=True),
         jnp.sum(y * y, axis=0, keepdims=True)], axis=0)

    @pl.when(pl.program_id(0) == 0)
    def _():
        st_ref[...] = st

    @pl.when(pl.program_id(0) != 0)
    def _():
        st_ref[...] = st_ref[...] + st


def _mm_stats(flat_feats, w_fa):
    bn_tot, ci = flat_feats.shape
    cw = w_fa.shape[1]
    blk = 2048
    grid = bn_tot // blk
    return pl.pallas_call(
        _mm_stats_body,
        grid=(grid,),
        in_specs=[
            pl.BlockSpec((blk, ci), lambda i: (i, 0)),
            pl.BlockSpec((ci, cw), lambda i: (0, 0)),
        ],
        out_specs=[
            pl.BlockSpec((blk, cw), lambda i: (i, 0)),
            pl.BlockSpec((2, cw), lambda i: (0, 0)),
        ],
        out_shape=[
            jax.ShapeDtypeStruct((bn_tot, cw), jnp.float32),
            jax.ShapeDtypeStruct((2, cw), jnp.float32),
        ],
    )(flat_feats, w_fa)


def _ball_query_sc(xs, ys, zs, cxs, cys, czs, B, N, M):
    BM = B * M
    CPT = BM // NW
    TPB = NW // B
    MPT = M // TPB
    nchunk = N // L
    r2 = jnp.float32(RADIUS * RADIUS)

    mesh = plsc.VectorSubcoreMesh(
        core_axis_name="c", subcore_axis_name="s", num_cores=NC,
        num_subcores=NS)

    @functools.partial(
        pl.kernel,
        mesh=mesh,
        compiler_params=pltpu.CompilerParams(needs_layout_passes=False),
        out_type=[
            jax.ShapeDtypeStruct((BM * K,), jnp.int32),
            jax.ShapeDtypeStruct((BM,), jnp.int32),
            jax.ShapeDtypeStruct((BM * 3 * K,), jnp.float32),
            jax.ShapeDtypeStruct((NW * 16 * L,), jnp.float32),
        ],
        scratch_types=[
            pltpu.VMEM((N,), jnp.float32),
            pltpu.VMEM((N,), jnp.float32),
            pltpu.VMEM((N,), jnp.float32),
            pltpu.VMEM((MPT + L,), jnp.float32),
            pltpu.VMEM((MPT + L,), jnp.float32),
            pltpu.VMEM((MPT + L,), jnp.float32),
            pltpu.VMEM((2 * (K + L),), jnp.int32),
            pltpu.VMEM((CPT * K,), jnp.int32),
            pltpu.VMEM((CPT,), jnp.int32),
            pltpu.VMEM((CPT * 3 * K,), jnp.float32),
            pltpu.VMEM((16 * L,), jnp.float32),
        ],
    )
    def kb(x_hbm, y_hbm, z_hbm, cx_hbm, cy_hbm, cz_hbm,
           idx_out, cnt_out, off_out, part_out,
           x_v, y_v, z_v, cx_v, cy_v, cz_v, ib_v, idx_t, cnt_t, off_t, acc_v):
        wid = lax.axis_index("s") * NC + lax.axis_index("c")
        b = wid // TPB
        mb = b * M + (wid % TPB) * MPT
        pltpu.sync_copy(x_hbm.at[pl.ds(b * N, N)], x_v)
        pltpu.sync_copy(y_hbm.at[pl.ds(b * N, N)], y_v)
        pltpu.sync_copy(z_hbm.at[pl.ds(b * N, N)], z_v)
        pltpu.sync_copy(cx_hbm.at[pl.ds(mb, MPT)], cx_v.at[pl.ds(0, MPT)])
        pltpu.sync_copy(cy_hbm.at[pl.ds(mb, MPT)], cy_v.at[pl.ds(0, MPT)])
        pltpu.sync_copy(cz_hbm.at[pl.ds(mb, MPT)], cz_v.at[pl.ds(0, MPT)])

        lanes = lax.iota(jnp.int32, L)
        for r in range(16):
            acc_v[pl.ds(r * L, L)] = jnp.zeros((L,), jnp.float32)
        row_base = jnp.int32(b * N)

        def per_pair(p, _):
            i0 = 2 * p
            i1 = 2 * p + 1
            cs = [(cx_v[pl.ds(i, L)][0], cy_v[pl.ds(i, L)][0],
                   cz_v[pl.ds(i, L)][0]) for i in (i0, i1)]
            ibs = [0, K + L]

            def cond(st):
                j, c0, c1 = st
                return jnp.logical_and(j < nchunk,
                                       jnp.logical_or(c0 < K, c1 < K))

            def body(st):
                j, c0, c1 = st
                cc = [c0, c1]
                for u in range(4):
                    base = (j + u) * L
                    xs_ = x_v[pl.ds(base, L)]
                    ys_ = y_v[pl.ds(base, L)]
                    zs_ = z_v[pl.ds(base, L)]
                    for t in range(2):
                        dx = xs_ - cs[t][0]
                        dy = ys_ - cs[t][1]
                        dz = zs_ - cs[t][2]
                        d2 = dx * dx + dy * dy + dz * dz
                        msk = d2 < r2
                        plsc.store_compressed(
                            ib_v.at[pl.ds(ibs[t] + cc[t], L)],
                            base + lanes, mask=msk)
                        pc = plsc.all_reduce_population_count(msk)
                        cc[t] = jnp.minimum(cc[t] + pc[0], K)
                return j + 4, cc[0], cc[1]

            _, c0, c1 = lax.while_loop(
                cond, body, (jnp.int32(0), jnp.int32(0), jnp.int32(0)))

            for t, (i, c) in enumerate(((i0, c0), (i1, c1))):
                cxi, cyi, czi = cs[t]
                ibb = ibs[t]
                plsc.store_scatter(cnt_t, [jnp.full((L,), i, jnp.int32)],
                                   jnp.full((L,), c, jnp.int32),
                                   mask=lanes < 1)
                first = jnp.where(c > 0, ib_v[pl.ds(ibb, L)][0], 0)
                for v in range(K // L):
                    ln = jnp.int32(v * L) + lanes
                    vals = ib_v[pl.ds(ibb + v * L, L)]
                    vals = jnp.where(ln < c, vals, first)
                    ox = plsc.load_gather(x_v, [vals]) - cxi
                    oy = plsc.load_gather(y_v, [vals]) - cyi
                    oz = plsc.load_gather(z_v, [vals]) - czi
                    ob = i * (3 * K) + v * L
                    off_t[pl.ds(ob, L)] = ox
                    off_t[pl.ds(ob + K, L)] = oy
                    off_t[pl.ds(ob + 2 * K, L)] = oz
                    acc_v[pl.ds(0 * L, L)] = acc_v[pl.ds(0 * L, L)] + ox
                    acc_v[pl.ds(1 * L, L)] = acc_v[pl.ds(1 * L, L)] + oy
                    acc_v[pl.ds(2 * L, L)] = acc_v[pl.ds(2 * L, L)] + oz
                    acc_v[pl.ds(3 * L, L)] = acc_v[pl.ds(3 * L, L)] + ox * ox
                    acc_v[pl.ds(4 * L, L)] = acc_v[pl.ds(4 * L, L)] + oy * oy
                    acc_v[pl.ds(5 * L, L)] = acc_v[pl.ds(5 * L, L)] + oz * oz
                    acc_v[pl.ds(6 * L, L)] = acc_v[pl.ds(6 * L, L)] + ox * oy
                    acc_v[pl.ds(7 * L, L)] = acc_v[pl.ds(7 * L, L)] + ox * oz
                    acc_v[pl.ds(8 * L, L)] = acc_v[pl.ds(8 * L, L)] + oy * oz
                    idx_t[pl.ds(i * K + v * L, L)] = vals + row_base
            return 0

        lax.fori_loop(0, CPT // 2, per_pair, 0)
        pltpu.sync_copy(idx_t, idx_out.at[pl.ds(wid * (CPT * K), CPT * K)])
        pltpu.sync_copy(cnt_t, cnt_out.at[pl.ds(wid * CPT, CPT)])
        pltpu.sync_copy(off_t,
                        off_out.at[pl.ds(wid * (CPT * 3 * K), CPT * 3 * K)])
        pltpu.sync_copy(acc_v, part_out.at[pl.ds(wid * (16 * L), 16 * L)])

    return kb(xs, ys, zs, cxs, cys, czs)


def _finalize_body(n1, n2, st_ref, part_ref, gfa_ref, bfa_ref, wx_ref,
                   gx_ref, bx_ref, out_ref):
    st = st_ref[...]
    mean_fa = st[0:1, :] / n1
    var_fa = st[1:2, :] / n1 - mean_fa * mean_fa
    scale_fa = gfa_ref[...] * lax.rsqrt(var_fa + EPS)
    shift_fa = bfa_ref[...] - mean_fa * scale_fa

    p = part_ref[...]
    s = jnp.sum(jnp.sum(p, axis=0), axis=1, keepdims=True)
    w0 = wx_ref[0:1, :]
    w1 = wx_ref[1:2, :]
    w2 = wx_ref[2:3, :]
    mx = (s[0, 0] * w0 + s[1, 0] * w1 + s[2, 0] * w2) / n2
    exx = (s[3, 0] * w0 * w0 + s[4, 0] * w1 * w1 + s[5, 0] * w2 * w2
           + 2.0 * (s[6, 0] * w0 * w1 + s[7, 0] * w0 * w2
                    + s[8, 0] * w1 * w2)) / n2
    var_x = exx - mx * mx
    scale_x = gx_ref[...] * lax.rsqrt(var_x + EPS)
    shift_x = bx_ref[...] - mx * scale_x

    pad = jnp.zeros_like(w0)
    xrows = jnp.concatenate(
        [jnp.concatenate([r * scale_x, pad], axis=1)
         for r in (w0, w1, w2)] + [jnp.concatenate([shift_x, pad], axis=1)],
        axis=0)
    out_ref[...] = jnp.concatenate(
        [scale_fa, shift_fa, xrows,
         jnp.zeros((2, scale_fa.shape[1]), jnp.float32)], axis=0)


def _finalize(st, part, g_fa, b_fa, w_xyz, g_xyz, b_xyz, n1, n2):
    cw = st.shape[1]
    c0 = w_xyz.shape[1]
    return pl.pallas_call(
        functools.partial(_finalize_body, float(n1), float(n2)),
        out_shape=jax.ShapeDtypeStruct((8, cw), jnp.float32),
    )(st, part, g_fa.reshape(1, cw), b_fa.reshape(1, cw), w_xyz,
      g_xyz.reshape(1, c0), b_xyz.reshape(1, c0))


def _attend_sc(yfa, idx_flat, off_flat, consts, BM, C0):
    CW = yfa.shape[1]
    CPT = BM // NW
    G = 8
    NG = CPT // G
    NCV = C0 // L

    mesh = plsc.VectorSubcoreMesh(
        core_axis_name="c", subcore_axis_name="s", num_cores=NC,
        num_subcores=NS)

    @functools.partial(
        pl.kernel,
        mesh=mesh,
        compiler_params=pltpu.CompilerParams(needs_layout_passes=False),
        out_type=jax.ShapeDtypeStruct((BM * C0,), jnp.float32),
        scratch_types=[
            pltpu.VMEM((CPT * K,), jnp.int32),
            pltpu.VMEM((CPT * 3 * K,), jnp.float32),
            pltpu.VMEM((G * K, CW), jnp.float32),
            pltpu.VMEM((G * K, CW), jnp.float32),
            pltpu.VMEM((8, CW), jnp.float32),
            pltpu.VMEM((CPT * C0,), jnp.float32),
            pltpu.SemaphoreType.DMA,
            pltpu.SemaphoreType.DMA,
        ],
    )
    def kd(yfa_hbm, idx_hbm, off_hbm, c_hbm, out_hbm,
           idx_v, off_v, yb0_v, yb1_v, cc_v, out_v, sem0, sem1):
        wid = lax.axis_index("s") * NC + lax.axis_index("c")
        base = wid * CPT
        pltpu.sync_copy(idx_hbm.at[pl.ds(base * K, CPT * K)], idx_v)
        pltpu.sync_copy(off_hbm.at[pl.ds(base * 3 * K, CPT * 3 * K)], off_v)
        pltpu.sync_copy(c_hbm, cc_v)

        sc = [cc_v[0, pl.ds(cv * L, L)] for cv in range(2 * NCV)]
        sh = [cc_v[1, pl.ds(cv * L, L)] for cv in range(2 * NCV)]
        we0 = [cc_v[2, pl.ds(cv * L, L)] for cv in range(NCV)]
        we1 = [cc_v[3, pl.ds(cv * L, L)] for cv in range(NCV)]
        we2 = [cc_v[4, pl.ds(cv * L, L)] for cv in range(NCV)]
        hx = [cc_v[5, pl.ds(cv * L, L)] for cv in range(NCV)]
        GK = G * K

        def gsrc(g):
            return yfa_hbm.at[idx_v.at[pl.ds(g * GK, GK)]]

        def compute_group(g, buf):
            def per_centroid(ci, _):
                r = g * G + ci
                ob = r * (3 * K)
                dq = [off_v[pl.ds(ob + q * L, L)] for q in range(6)]
                rowb = ci * K
                nums = [jnp.zeros((L,), jnp.float32) for _ in range(NCV)]
                dens = [jnp.zeros((L,), jnp.float32) for _ in range(NCV)]
                for k in range(K):
                    q, ln = divmod(k, L)
                    dxk = dq[q][ln]
                    dyk = dq[2 + q][ln]
                    dzk = dq[4 + q][ln]
                    row = rowb + k
                    for cv in range(NCV):
                        s = pl.ds(cv * L, L)
                        sa = pl.ds(C0 + cv * L, L)
                        gxf = (dxk * we0[cv] + dyk * we1[cv]
                               + dzk * we2[cv] + hx[cv])
                        f = buf[row, s] * sc[cv] + sh[cv] + gxf
                        a = (buf[row, sa] * sc[NCV + cv]
                             + sh[NCV + cv] + gxf)
                        e = jnp.exp(f)
                        nums[cv] = nums[cv] + e * a
                        dens[cv] = dens[cv] + e
                for cv in range(NCV):
                    out_v[pl.ds(r * C0 + cv * L, L)] = nums[cv] / dens[cv]
                return 0

            lax.fori_loop(0, G, per_centroid, 0)

        pltpu.async_copy(gsrc(0), yb0_v, sem0)

        def per_pair(p, _):
            g0 = 2 * p
            pltpu.make_async_copy(gsrc(g0), yb0_v, sem0).wait()
            pltpu.async_copy(gsrc(g0 + 1), yb1_v, sem1)
            compute_group(g0, yb0_v)
            pltpu.make_async_copy(gsrc(g0 + 1), yb1_v, sem1).wait()

            @pl.when(p + 1 < NG // 2)
            def _():
                pltpu.async_copy(gsrc(g0 + 2), yb0_v, sem0)

            compute_group(g0 + 1, yb1_v)
            return 0

        lax.fori_loop(0, NG // 2, per_pair, 0)
        pltpu.sync_copy(out_v, out_hbm.at[pl.ds(base * C0, CPT * C0)])

    return kd(yfa, idx_flat, off_flat, consts)


def _mlp_body(x_ref, cnt_ref, w1_ref, g1_ref, b1_ref, w2_ref, g2_ref, b2_ref,
              out_ref):
    x = x_ref[...]
    n = x.shape[0]
    t = jnp.dot(x, w1_ref[...], preferred_element_type=jnp.float32)
    m = jnp.sum(t, axis=0, keepdims=True) / n
    v = jnp.sum(t * t, axis=0, keepdims=True) / n - m * m
    t = (t - m) * lax.rsqrt(v + EPS) * g1_ref[...] + b1_ref[...]
    t = jnp.maximum(t, 0.0)
    u = jnp.dot(t, w2_ref[...], preferred_element_type=jnp.float32)
    m2 = jnp.sum(u, axis=0, keepdims=True) / n
    v2 = jnp.sum(u * u, axis=0, keepdims=True) / n - m2 * m2
    u = (u - m2) * lax.rsqrt(v2 + EPS) * g2_ref[...] + b2_ref[...]
    u = jnp.maximum(u, 0.0)
    valid = (cnt_ref[...] > 0).astype(jnp.float32)
    out_ref[...] = u * valid


def _mlp(nf, cnt, w1, g1, b1, w2, g2, b2):
    bm, c1 = nf.shape[0], w1.shape[1]
    c2 = w2.shape[1]
    return pl.pallas_call(
        _mlp_body,
        out_shape=jax.ShapeDtypeStruct((bm, c2), jnp.float32),
    )(nf, cnt.reshape(bm, 1), w1, g1.reshape(1, c1), b1.reshape(1, c1),
      w2, g2.reshape(1, c2), b2.reshape(1, c2))


def kernel(xyz, new_xyz, feats, W_feat, g_feat, b_feat, W_attn, g_attn,
           b_attn, W_xyz, g_xyz, b_xyz, W_m1, g_m1, b_m1, W_m2, g_m2, b_m2):
    B, N, _ = xyz.shape
    M = new_xyz.shape[1]
    Ci = feats.shape[2]
    C0 = W_feat.shape[1]
    C2 = W_m2.shape[1]
    BM = B * M

    xs = xyz[:, :, 0].reshape(-1)
    ys = xyz[:, :, 1].reshape(-1)
    zs = xyz[:, :, 2].reshape(-1)
    cxs = new_xyz[:, :, 0].reshape(-1)
    cys = new_xyz[:, :, 1].reshape(-1)
    czs = new_xyz[:, :, 2].reshape(-1)

    w_fa = jnp.concatenate([W_feat, W_attn], axis=1)
    g_fa = jnp.concatenate([g_feat, g_attn], axis=0)
    b_fa = jnp.concatenate([b_feat, b_attn], axis=0)

    yfa, st = _mm_stats(feats.reshape(B * N, Ci), w_fa)
    idx, cnt, off, part = _ball_query_sc(xs, ys, zs, cxs, cys, czs, B, N, M)
    consts = _finalize(st, part.reshape(NW, 16, L), g_fa, b_fa, W_xyz,
                       g_xyz, b_xyz, B * N, BM * K)
    nf = _attend_sc(yfa, idx, off, consts, BM, C0)
    out = _mlp(nf.reshape(BM, C0), cnt, W_m1, g_m1, b_m1, W_m2, g_m2, b_m2)
    return out.reshape(B, M, C2)


# speedup vs baseline: 1.6823x; 1.0577x over previous
import functools

import jax
import jax.numpy as jnp
from jax import lax
from jax.experimental import pallas as pl
from jax.experimental.pallas import tpu as pltpu
from jax.experimental.pallas import tpu_sc as plsc

RADIUS = 0.2
K = 32
EPS = 1e-5

NC = 2
NS = 16
L = 16
NW = NC * NS


def _mm_stats_body(x_ref, w_ref, y_ref, st_ref):
    x = x_ref[...]
    y = jnp.dot(x, w_ref[...], preferred_element_type=jnp.float32)
    y_ref[...] = y
    st = jnp.concatenate(
        [jnp.sum(y, axis=0, keepdims=True),
         jnp.sum(y * y, axis=0, keepdims=True)], axis=0)

    @pl.when(pl.program_id(0) == 0)
    def _():
        st_ref[...] = st

    @pl.when(pl.program_id(0) != 0)
    def _():
        st_ref[...] = st_ref[...] + st


def _mm_stats(flat_feats, w_fa):
    bn_tot, ci = flat_feats.shape
    cw = w_fa.shape[1]
    blk = 2048
    grid = bn_tot // blk
    return pl.pallas_call(
        _mm_stats_body,
        grid=(grid,),
        in_specs=[
            pl.BlockSpec((blk, ci), lambda i: (i, 0)),
            pl.BlockSpec((ci, cw), lambda i: (0, 0)),
        ],
        out_specs=[
            pl.BlockSpec((blk, cw), lambda i: (i, 0)),
            pl.BlockSpec((2, cw), lambda i: (0, 0)),
        ],
        out_shape=[
            jax.ShapeDtypeStruct((bn_tot, cw), jnp.float32),
            jax.ShapeDtypeStruct((2, cw), jnp.float32),
        ],
    )(flat_feats, w_fa)


def _ball_query_sc(xs, ys, zs, cxs, cys, czs, B, N, M):
    BM = B * M
    CPT = BM // NW
    TPB = NW // B
    MPT = M // TPB
    nchunk = N // L
    r2 = jnp.float32(RADIUS * RADIUS)

    mesh = plsc.VectorSubcoreMesh(
        core_axis_name="c", subcore_axis_name="s", num_cores=NC,
        num_subcores=NS)

    @functools.partial(
        pl.kernel,
        mesh=mesh,
        compiler_params=pltpu.CompilerParams(needs_layout_passes=False),
        out_type=[
            jax.ShapeDtypeStruct((BM * K,), jnp.int32),
            jax.ShapeDtypeStruct((BM,), jnp.int32),
            jax.ShapeDtypeStruct((BM * 3 * K,), jnp.float32),
            jax.ShapeDtypeStruct((NW * 16 * L,), jnp.float32),
        ],
        scratch_types=[
            pltpu.VMEM((N,), jnp.float32),
            pltpu.VMEM((N,), jnp.float32),
            pltpu.VMEM((N,), jnp.float32),
            pltpu.VMEM((MPT + L,), jnp.float32),
            pltpu.VMEM((MPT + L,), jnp.float32),
            pltpu.VMEM((MPT + L,), jnp.float32),
            pltpu.VMEM((4 * (K + L),), jnp.int32),
            pltpu.VMEM((CPT * K,), jnp.int32),
            pltpu.VMEM((CPT,), jnp.int32),
            pltpu.VMEM((CPT * 3 * K,), jnp.float32),
            pltpu.VMEM((16 * L,), jnp.float32),
        ],
    )
    def kb(x_hbm, y_hbm, z_hbm, cx_hbm, cy_hbm, cz_hbm,
           idx_out, cnt_out, off_out, part_out,
           x_v, y_v, z_v, cx_v, cy_v, cz_v, ib_v, idx_t, cnt_t, off_t, acc_v):
        wid = lax.axis_index("s") * NC + lax.axis_index("c")
        b = wid // TPB
        mb = b * M + (wid % TPB) * MPT
        pltpu.sync_copy(x_hbm.at[pl.ds(b * N, N)], x_v)
        pltpu.sync_copy(y_hbm.at[pl.ds(b * N, N)], y_v)
        pltpu.sync_copy(z_hbm.at[pl.ds(b * N, N)], z_v)
        pltpu.sync_copy(cx_hbm.at[pl.ds(mb, MPT)], cx_v.at[pl.ds(0, MPT)])
        pltpu.sync_copy(cy_hbm.at[pl.ds(mb, MPT)], cy_v.at[pl.ds(0, MPT)])
        pltpu.sync_copy(cz_hbm.at[pl.ds(mb, MPT)], cz_v.at[pl.ds(0, MPT)])

        lanes = lax.iota(jnp.int32, L)
        for r in range(16):
            acc_v[pl.ds(r * L, L)] = jnp.zeros((L,), jnp.float32)
        row_base = jnp.int32(b * N)

        def per_quad(p, _):
            iq = [4 * p + t for t in range(4)]
            cs = [(cx_v[pl.ds(i, L)][0], cy_v[pl.ds(i, L)][0],
                   cz_v[pl.ds(i, L)][0]) for i in iq]
            ibs = [t * (K + L) for t in range(4)]

            def cond(st):
                j = st[0]
                done = st[1] >= K
                for t in range(2, 5):
                    done = jnp.logical_and(done, st[t] >= K)
                return jnp.logical_and(j < nchunk, jnp.logical_not(done))

            def body(st):
                j = st[0]
                cc = list(st[1:])
                for u in range(2):
                    base = (j + u) * L
                    xs_ = x_v[pl.ds(base, L)]
                    ys_ = y_v[pl.ds(base, L)]
                    zs_ = z_v[pl.ds(base, L)]
                    for t in range(4):
                        dx = xs_ - cs[t][0]
                        dy = ys_ - cs[t][1]
                        dz = zs_ - cs[t][2]
                        d2 = dx * dx + dy * dy + dz * dz
                        msk = d2 < r2
                        plsc.store_compressed(
                            ib_v.at[pl.ds(ibs[t] + cc[t], L)],
                            base + lanes, mask=msk)
                        pc = plsc.all_reduce_population_count(msk)
                        cc[t] = jnp.minimum(cc[t] + pc[0], K)
                return (j + 2,) + tuple(cc)

            fin = lax.while_loop(
                cond, body, (jnp.int32(0),) + (jnp.int32(0),) * 4)

            for t, (i, c) in enumerate(zip(iq, fin[1:])):
                cxi, cyi, czi = cs[t]
                ibb = ibs[t]
                plsc.store_scatter(cnt_t, [jnp.full((L,), i, jnp.int32)],
                                   jnp.full((L,), c, jnp.int32),
                                   mask=lanes < 1)
                first = jnp.where(c > 0, ib_v[pl.ds(ibb, L)][0], 0)
                for v in range(K // L):
                    ln = jnp.int32(v * L) + lanes
                    vals = ib_v[pl.ds(ibb + v * L, L)]
                    vals = jnp.where(ln < c, vals, first)
                    ox = plsc.load_gather(x_v, [vals]) - cxi
                    oy = plsc.load_gather(y_v, [vals]) - cyi
                    oz = plsc.load_gather(z_v, [vals]) - czi
                    ob = i * (3 * K) + v * L
                    off_t[pl.ds(ob, L)] = ox
                    off_t[pl.ds(ob + K, L)] = oy
                    off_t[pl.ds(ob + 2 * K, L)] = oz
                    acc_v[pl.ds(0 * L, L)] = acc_v[pl.ds(0 * L, L)] + ox
                    acc_v[pl.ds(1 * L, L)] = acc_v[pl.ds(1 * L, L)] + oy
                    acc_v[pl.ds(2 * L, L)] = acc_v[pl.ds(2 * L, L)] + oz
                    acc_v[pl.ds(3 * L, L)] = acc_v[pl.ds(3 * L, L)] + ox * ox
                    acc_v[pl.ds(4 * L, L)] = acc_v[pl.ds(4 * L, L)] + oy * oy
                    acc_v[pl.ds(5 * L, L)] = acc_v[pl.ds(5 * L, L)] + oz * oz
                    acc_v[pl.ds(6 * L, L)] = acc_v[pl.ds(6 * L, L)] + ox * oy
                    acc_v[pl.ds(7 * L, L)] = acc_v[pl.ds(7 * L, L)] + ox * oz
                    acc_v[pl.ds(8 * L, L)] = acc_v[pl.ds(8 * L, L)] + oy * oz
                    idx_t[pl.ds(i * K + v * L, L)] = vals + row_base
            return 0

        lax.fori_loop(0, CPT // 4, per_quad, 0)
        pltpu.sync_copy(idx_t, idx_out.at[pl.ds(wid * (CPT * K), CPT * K)])
        pltpu.sync_copy(cnt_t, cnt_out.at[pl.ds(wid * CPT, CPT)])
        pltpu.sync_copy(off_t,
                        off_out.at[pl.ds(wid * (CPT * 3 * K), CPT * 3 * K)])
        pltpu.sync_copy(acc_v, part_out.at[pl.ds(wid * (16 * L), 16 * L)])

    return kb(xs, ys, zs, cxs, cys, czs)


def _finalize_body(n1, n2, st_ref, part_ref, gfa_ref, bfa_ref, wx_ref,
                   gx_ref, bx_ref, out_ref):
    st = st_ref[...]
    mean_fa = st[0:1, :] / n1
    var_fa = st[1:2, :] / n1 - mean_fa * mean_fa
    scale_fa = gfa_ref[...] * lax.rsqrt(var_fa + EPS)
    shift_fa = bfa_ref[...] - mean_fa * scale_fa

    p = part_ref[...]
    s = jnp.sum(jnp.sum(p, axis=0), axis=1, keepdims=True)
    w0 = wx_ref[0:1, :]
    w1 = wx_ref[1:2, :]
    w2 = wx_ref[2:3, :]
    mx = (s[0, 0] * w0 + s[1, 0] * w1 + s[2, 0] * w2) / n2
    exx = (s[3, 0] * w0 * w0 + s[4, 0] * w1 * w1 + s[5, 0] * w2 * w2
           + 2.0 * (s[6, 0] * w0 * w1 + s[7, 0] * w0 * w2
                    + s[8, 0] * w1 * w2)) / n2
    var_x = exx - mx * mx
    scale_x = gx_ref[...] * lax.rsqrt(var_x + EPS)
    shift_x = bx_ref[...] - mx * scale_x

    pad = jnp.zeros_like(w0)
    xrows = jnp.concatenate(
        [jnp.concatenate([r * scale_x, pad], axis=1)
         for r in (w0, w1, w2)] + [jnp.concatenate([shift_x, pad], axis=1)],
        axis=0)
    out_ref[...] = jnp.concatenate(
        [scale_fa, shift_fa, xrows,
         jnp.zeros((2, scale_fa.shape[1]), jnp.float32)], axis=0)


def _finalize(st, part, g_fa, b_fa, w_xyz, g_xyz, b_xyz, n1, n2):
    cw = st.shape[1]
    c0 = w_xyz.shape[1]
    return pl.pallas_call(
        functools.partial(_finalize_body, float(n1), float(n2)),
        out_shape=jax.ShapeDtypeStruct((8, cw), jnp.float32),
    )(st, part, g_fa.reshape(1, cw), b_fa.reshape(1, cw), w_xyz,
      g_xyz.reshape(1, c0), b_xyz.reshape(1, c0))


def _attend_sc(yfa, idx_flat, off_flat, consts, BM, C0):
    CW = yfa.shape[1]
    CPT = BM // NW
    G = 8
    NG = CPT // G
    NCV = C0 // L

    mesh = plsc.VectorSubcoreMesh(
        core_axis_name="c", subcore_axis_name="s", num_cores=NC,
        num_subcores=NS)

    @functools.partial(
        pl.kernel,
        mesh=mesh,
        compiler_params=pltpu.CompilerParams(needs_layout_passes=False),
        out_type=jax.ShapeDtypeStruct((BM * C0,), jnp.float32),
        scratch_types=[
            pltpu.VMEM((CPT * K,), jnp.int32),
            pltpu.VMEM((CPT * 3 * K,), jnp.float32),
            pltpu.VMEM((G * K, CW), jnp.float32),
            pltpu.VMEM((G * K, CW), jnp.float32),
            pltpu.VMEM((8, CW), jnp.float32),
            pltpu.VMEM((CPT * C0,), jnp.float32),
            pltpu.SemaphoreType.DMA,
            pltpu.SemaphoreType.DMA,
        ],
    )
    def kd(yfa_hbm, idx_hbm, off_hbm, c_hbm, out_hbm,
           idx_v, off_v, yb0_v, yb1_v, cc_v, out_v, sem0, sem1):
        wid = lax.axis_index("s") * NC + lax.axis_index("c")
        base = wid * CPT
        pltpu.sync_copy(idx_hbm.at[pl.ds(base * K, CPT * K)], idx_v)
        pltpu.sync_copy(off_hbm.at[pl.ds(base * 3 * K, CPT * 3 * K)], off_v)
        pltpu.sync_copy(c_hbm, cc_v)

        sc = [cc_v[0, pl.ds(cv * L, L)] for cv in range(2 * NCV)]
        sh = [cc_v[1, pl.ds(cv * L, L)] for cv in range(2 * NCV)]
        we0 = [cc_v[2, pl.ds(cv * L, L)] for cv in range(NCV)]
        we1 = [cc_v[3, pl.ds(cv * L, L)] for cv in range(NCV)]
        we2 = [cc_v[4, pl.ds(cv * L, L)] for cv in range(NCV)]
        hx = [cc_v[5, pl.ds(cv * L, L)] for cv in range(NCV)]
        GK = G * K

        def gsrc(g):
            return yfa_hbm.at[idx_v.at[pl.ds(g * GK, GK)]]

        def compute_group(g, buf):
            def per_centroid(ci, _):
                r = g * G + ci
                ob = r * (3 * K)
                dq = [off_v[pl.ds(ob + q * L, L)] for q in range(6)]
                rowb = ci * K
                nums = [jnp.zeros((L,), jnp.float32) for _ in range(NCV)]
                dens = [jnp.zeros((L,), jnp.float32) for _ in range(NCV)]
                for k in range(K):
                    q, ln = divmod(k, L)
                    dxk = dq[q][ln]
                    dyk = dq[2 + q][ln]
                    dzk = dq[4 + q][ln]
                    row = rowb + k
                    for cv in range(NCV):
                        s = pl.ds(cv * L, L)
                        sa = pl.ds(C0 + cv * L, L)
                        gxf = (dxk * we0[cv] + dyk * we1[cv]
                               + dzk * we2[cv] + hx[cv])
                        f = buf[row, s] * sc[cv] + sh[cv] + gxf
                        a = (buf[row, sa] * sc[NCV + cv]
                             + sh[NCV + cv] + gxf)
                        e = jnp.exp(f)
                        nums[cv] = nums[cv] + e * a
                        dens[cv] = dens[cv] + e
                for cv in range(NCV):
                    out_v[pl.ds(r * C0 + cv * L, L)] = nums[cv] / dens[cv]
                return 0

            lax.fori_loop(0, G, per_centroid, 0)

        pltpu.async_copy(gsrc(0), yb0_v, sem0)

        def per_pair(p, _):
            g0 = 2 * p
            pltpu.make_async_copy(gsrc(g0), yb0_v, sem0).wait()
            pltpu.async_copy(gsrc(g0 + 1), yb1_v, sem1)
            compute_group(g0, yb0_v)
            pltpu.make_async_copy(gsrc(g0 + 1), yb1_v, sem1).wait()

            @pl.when(p + 1 < NG // 2)
            def _():
                pltpu.async_copy(gsrc(g0 + 2), yb0_v, sem0)

            compute_group(g0 + 1, yb1_v)
            return 0

        lax.fori_loop(0, NG // 2, per_pair, 0)
        pltpu.sync_copy(out_v, out_hbm.at[pl.ds(base * C0, CPT * C0)])

    return kd(yfa, idx_flat, off_flat, consts)


def _mlp_body(x_ref, cnt_ref, w1_ref, g1_ref, b1_ref, w2_ref, g2_ref, b2_ref,
              out_ref):
    x = x_ref[...]
    n = x.shape[0]
    t = jnp.dot(x, w1_ref[...], preferred_element_type=jnp.float32)
    m = jnp.sum(t, axis=0, keepdims=True) / n
    v = jnp.sum(t * t, axis=0, keepdims=True) / n - m * m
    t = (t - m) * lax.rsqrt(v + EPS) * g1_ref[...] + b1_ref[...]
    t = jnp.maximum(t, 0.0)
    u = jnp.dot(t, w2_ref[...], preferred_element_type=jnp.float32)
    m2 = jnp.sum(u, axis=0, keepdims=True) / n
    v2 = jnp.sum(u * u, axis=0, keepdims=True) / n - m2 * m2
    u = (u - m2) * lax.rsqrt(v2 + EPS) * g2_ref[...] + b2_ref[...]
    u = jnp.maximum(u, 0.0)
    valid = (cnt_ref[...] > 0).astype(jnp.float32)
    out_ref[...] = u * valid


def _mlp(nf, cnt, w1, g1, b1, w2, g2, b2):
    bm, c1 = nf.shape[0], w1.shape[1]
    c2 = w2.shape[1]
    return pl.pallas_call(
        _mlp_body,
        out_shape=jax.ShapeDtypeStruct((bm, c2), jnp.float32),
    )(nf, cnt.reshape(bm, 1), w1, g1.reshape(1, c1), b1.reshape(1, c1),
      w2, g2.reshape(1, c2), b2.reshape(1, c2))


def kernel(xyz, new_xyz, feats, W_feat, g_feat, b_feat, W_attn, g_attn,
           b_attn, W_xyz, g_xyz, b_xyz, W_m1, g_m1, b_m1, W_m2, g_m2, b_m2):
    B, N, _ = xyz.shape
    M = new_xyz.shape[1]
    Ci = feats.shape[2]
    C0 = W_feat.shape[1]
    C2 = W_m2.shape[1]
    BM = B * M

    xs = xyz[:, :, 0].reshape(-1)
    ys = xyz[:, :, 1].reshape(-1)
    zs = xyz[:, :, 2].reshape(-1)
    cxs = new_xyz[:, :, 0].reshape(-1)
    cys = new_xyz[:, :, 1].reshape(-1)
    czs = new_xyz[:, :, 2].reshape(-1)

    w_fa = jnp.concatenate([W_feat, W_attn], axis=1)
    g_fa = jnp.concatenate([g_feat, g_attn], axis=0)
    b_fa = jnp.concatenate([b_feat, b_attn], axis=0)

    yfa, st = _mm_stats(feats.reshape(B * N, Ci), w_fa)
    idx, cnt, off, part = _ball_query_sc(xs, ys, zs, cxs, cys, czs, B, N, M)
    consts = _finalize(st, part.reshape(NW, 16, L), g_fa, b_fa, W_xyz,
                       g_xyz, b_xyz, B * N, BM * K)
    nf = _attend_sc(yfa, idx, off, consts, BM, C0)
    out = _mlp(nf.reshape(BM, C0), cnt, W_m1, g_m1, b_m1, W_m2, g_m2, b_m2)
    return out.reshape(B, M, C2)
